# Initial kernel scaffold; baseline (speedup 1.0000x reference)
import jax
import jax.numpy as jnp
from jax.experimental import pallas as pl


def kernel(h, edge_index, edge_weight):
    raise NotImplementedError("write your pallas kernel here")


# speedup vs baseline: 1.9822x; 1.9822x over previous
import functools

import jax
import jax.numpy as jnp
from jax import lax
from jax.experimental import pallas as pl
from jax.experimental.pallas import tpu as pltpu
from jax.experimental.pallas import tpu_sc as plsc

N_NODES = 10000
N_EDGES = 320000
D_FEAT = 128

NW = 32
E_PER_W = N_EDGES // NW
CHUNK = 80
STEPS = E_PER_W // CHUNK
LANES = 16
NREG = D_FEAT // LANES


def _mesh():
    return plsc.VectorSubcoreMesh(core_axis_name="c", subcore_axis_name="s")


@functools.partial(
    pl.kernel,
    out_type=jax.ShapeDtypeStruct((NW, LANES), jnp.float32),
    mesh=_mesh(),
    scratch_types=[
        pltpu.VMEM((CHUNK,), jnp.int32),
        pltpu.VMEM((CHUNK,), jnp.int32),
        pltpu.VMEM((CHUNK,), jnp.float32),
        pltpu.VMEM((CHUNK, D_FEAT), jnp.float32),
        pltpu.VMEM((CHUNK, D_FEAT), jnp.float32),
        pltpu.VMEM((LANES,), jnp.float32),
        pltpu.SemaphoreType.DMA,
        pltpu.SemaphoreType.DMA,
    ],
)
def _energy_kernel(src_hbm, dst_hbm, w_hbm, h_hbm, out_hbm,
                   sidx, didx, wbuf, srows, drows, accbuf, sem_s, sem_d):
    wid = lax.axis_index("s") * 2 + lax.axis_index("c")
    base0 = wid * E_PER_W

    def step(i, acc):
        base = pl.multiple_of(base0 + i * CHUNK, 8)
        pltpu.sync_copy(src_hbm.at[pl.ds(base, CHUNK)], sidx)
        pltpu.sync_copy(dst_hbm.at[pl.ds(base, CHUNK)], didx)
        pltpu.sync_copy(w_hbm.at[pl.ds(base, CHUNK)], wbuf)
        cps = pltpu.async_copy(h_hbm.at[sidx], srows, sem_s)
        cpd = pltpu.async_copy(h_hbm.at[didx], drows, sem_d)
        cps.wait()
        cpd.wait()

        def group(g, acc2):
            wv = wbuf[pl.ds(g * LANES, LANES)]
            for k in range(LANES):
                e = g * LANES + k
                sq = jnp.zeros((LANES,), jnp.float32)
                for j in range(NREG):
                    s = srows[e, pl.ds(j * LANES, LANES)]
                    d = drows[e, pl.ds(j * LANES, LANES)]
                    diff = s - d
                    sq = sq + diff * diff
                acc2 = acc2 + wv[k] * sq
            return acc2

        return lax.fori_loop(0, CHUNK // LANES, group, acc)

    acc = lax.fori_loop(0, STEPS, step, jnp.zeros((LANES,), jnp.float32))
    accbuf[...] = acc
    pltpu.sync_copy(accbuf, out_hbm.at[wid])


def kernel(h, edge_index, edge_weight):
    src = edge_index[0].astype(jnp.int32)
    dst = edge_index[1].astype(jnp.int32)
    partials = _energy_kernel(src, dst, edge_weight, h)
    return jnp.sum(partials) / N_EDGES


# speedup vs baseline: 5.8692x; 2.9610x over previous
import functools

import jax
import jax.numpy as jnp
from jax import lax
from jax.experimental import pallas as pl
from jax.experimental.pallas import tpu as pltpu
from jax.experimental.pallas import tpu_sc as plsc

N_NODES = 10000
N_EDGES = 320000
D_FEAT = 128

NW = 32
E_PER_W = N_EDGES // NW
CHUNK = 80
STEPS = E_PER_W // CHUNK
LANES = 16
NREG = D_FEAT // LANES
GROUPS = CHUNK // LANES


def _mesh():
    return plsc.VectorSubcoreMesh(core_axis_name="c", subcore_axis_name="s")


@functools.partial(
    pl.kernel,
    out_type=jax.ShapeDtypeStruct((NW, LANES), jnp.float32),
    mesh=_mesh(),
    scratch_types=[
        pltpu.VMEM((2, CHUNK), jnp.int32),
        pltpu.VMEM((2, CHUNK), jnp.int32),
        pltpu.VMEM((CHUNK,), jnp.float32),
        pltpu.VMEM((CHUNK,), jnp.float32),
        pltpu.VMEM((CHUNK, D_FEAT), jnp.float32),
        pltpu.VMEM((CHUNK, D_FEAT), jnp.float32),
        pltpu.VMEM((CHUNK, D_FEAT), jnp.float32),
        pltpu.VMEM((CHUNK, D_FEAT), jnp.float32),
        pltpu.VMEM((CHUNK, LANES), jnp.float32),
        pltpu.VMEM((LANES,), jnp.float32),
        pltpu.SemaphoreType.DMA,
        pltpu.SemaphoreType.DMA,
        pltpu.SemaphoreType.DMA,
        pltpu.SemaphoreType.DMA,
    ],
)
def _energy_kernel(packed_hbm, w_hbm, h_hbm, out_hbm,
                   ebuf0, ebuf1, wbuf0, wbuf1, srows0, srows1, drows0, drows1,
                   wsplat, accbuf, sem_s0, sem_s1, sem_d0, sem_d1):
    wid = lax.axis_index("s") * 2 + lax.axis_index("c")
    bufs = (
        (ebuf0, wbuf0, srows0, drows0, sem_s0, sem_d0),
        (ebuf1, wbuf1, srows1, drows1, sem_s1, sem_d1),
    )

    def start(i, b):
        ebuf, wbuf, srows, drows, sem_s, sem_d = bufs[b]
        pltpu.sync_copy(packed_hbm.at[wid, i], ebuf)
        pltpu.sync_copy(w_hbm.at[wid, i], wbuf)
        pltpu.async_copy(h_hbm.at[ebuf.at[0]], srows, sem_s)
        pltpu.async_copy(h_hbm.at[ebuf.at[1]], drows, sem_d)

    def wait(b):
        ebuf, wbuf, srows, drows, sem_s, sem_d = bufs[b]
        pltpu.make_async_copy(h_hbm.at[ebuf.at[0]], srows, sem_s).wait()
        pltpu.make_async_copy(h_hbm.at[ebuf.at[1]], drows, sem_d).wait()

    def compute(b, acc):
        ebuf, wbuf, srows, drows, _, _ = bufs[b]
        for g in range(GROUPS):
            wv = wbuf[pl.ds(g * LANES, LANES)]
            for k in range(LANES):
                wsplat[g * LANES + k, :] = jnp.broadcast_to(wv[k], (LANES,))

        def edge(e, acc2):
            w = wsplat[e, :]
            sq = jnp.zeros((LANES,), jnp.float32)
            for j in range(NREG):
                s = srows[e, pl.ds(j * LANES, LANES)]
                d = drows[e, pl.ds(j * LANES, LANES)]
                diff = s - d
                sq = sq + diff * diff
            return acc2 + w * sq

        return lax.fori_loop(0, CHUNK, edge, acc, unroll=2)

    acc = jnp.zeros((LANES,), jnp.float32)
    start(0, 0)

    def two_steps(g, acc2):
        start(2 * g + 1, 1)
        wait(0)
        acc2 = compute(0, acc2)
        start(2 * g + 2, 0)
        wait(1)
        return compute(1, acc2)

    acc = lax.fori_loop(0, (STEPS - 1) // 2, two_steps, acc)
    wait(0)
    acc = compute(0, acc)

    accbuf[...] = acc
    pltpu.sync_copy(accbuf, out_hbm.at[wid])


def kernel(h, edge_index, edge_weight):
    src = edge_index[0].astype(jnp.int32).reshape(NW, STEPS, CHUNK)
    dst = edge_index[1].astype(jnp.int32).reshape(NW, STEPS, CHUNK)
    packed = jnp.stack([src, dst], axis=2)
    wr = edge_weight.reshape(NW, STEPS, CHUNK)
    partials = _energy_kernel(packed, wr, h)
    return jnp.sum(partials) / N_EDGES


# speedup vs baseline: 7.9237x; 1.3500x over previous
import functools

import jax
import jax.numpy as jnp
from jax import lax
from jax.experimental import pallas as pl
from jax.experimental.pallas import tpu as pltpu
from jax.experimental.pallas import tpu_sc as plsc

N_NODES = 10000
N_EDGES = 320000
D_FEAT = 128

NW = 32
E_PER_W = N_EDGES // NW
CHUNK = 80
STEPS = E_PER_W // CHUNK
LANES = 16
NREG = D_FEAT // LANES
GROUPS = CHUNK // LANES
NBUF = 3
W_SCALE = float(1 << 24)


def _mesh():
    return plsc.VectorSubcoreMesh(core_axis_name="c", subcore_axis_name="s")


@functools.partial(
    pl.kernel,
    out_type=jax.ShapeDtypeStruct((NW, LANES), jnp.float32),
    mesh=_mesh(),
    scratch_types=(
        [pltpu.VMEM((3, CHUNK), jnp.int32) for _ in range(NBUF)]
        + [pltpu.VMEM((CHUNK, D_FEAT), jnp.float32) for _ in range(NBUF)]
        + [pltpu.VMEM((CHUNK, D_FEAT), jnp.float32) for _ in range(NBUF)]
        + [
            pltpu.VMEM((CHUNK, LANES), jnp.float32),
            pltpu.VMEM((LANES,), jnp.float32),
        ]
        + [pltpu.SemaphoreType.DMA for _ in range(3 * NBUF)]
    ),
)
def _energy_kernel(packed_hbm, h_hbm, out_hbm,
                   eb0, eb1, eb2, sr0, sr1, sr2, dr0, dr1, dr2,
                   wsplat, accbuf,
                   se0, se1, se2, ss0, ss1, ss2, sd0, sd1, sd2):
    wid = lax.axis_index("s") * 2 + lax.axis_index("c")
    ebufs = (eb0, eb1, eb2)
    srows = (sr0, sr1, sr2)
    drows = (dr0, dr1, dr2)
    sem_e = (se0, se1, se2)
    sem_s = (ss0, ss1, ss2)
    sem_d = (sd0, sd1, sd2)

    def start_ebuf(i, b):
        pltpu.async_copy(packed_hbm.at[wid, i], ebufs[b], sem_e[b])

    def wait_ebuf(i, b):
        pltpu.make_async_copy(packed_hbm.at[wid, i], ebufs[b], sem_e[b]).wait()

    def start_rows(b):
        pltpu.async_copy(h_hbm.at[ebufs[b].at[0]], srows[b], sem_s[b])
        pltpu.async_copy(h_hbm.at[ebufs[b].at[1]], drows[b], sem_d[b])

    def wait_rows(b):
        pltpu.make_async_copy(h_hbm.at[ebufs[b].at[0]], srows[b], sem_s[b]).wait()
        pltpu.make_async_copy(h_hbm.at[ebufs[b].at[1]], drows[b], sem_d[b]).wait()

    def compute(b, acc):
        eb, sr, dr = ebufs[b], srows[b], drows[b]
        for g in range(GROUPS):
            wv = eb[2, pl.ds(g * LANES, LANES)].astype(jnp.float32) * (1.0 / W_SCALE)
            for k in range(LANES):
                wsplat[g * LANES + k, :] = jnp.broadcast_to(wv[k], (LANES,))

        def edge(e, acc2):
            w = wsplat[e, :]
            sq = jnp.zeros((LANES,), jnp.float32)
            for j in range(NREG):
                s = sr[e, pl.ds(j * LANES, LANES)]
                d = dr[e, pl.ds(j * LANES, LANES)]
                diff = s - d
                sq = sq + diff * diff
            return acc2 + w * sq

        return lax.fori_loop(0, CHUNK, edge, acc, unroll=2)

    acc = jnp.zeros((LANES,), jnp.float32)

    start_ebuf(0, 0)
    start_ebuf(1, 1)
    wait_ebuf(0, 0)
    start_rows(0)

    def three_steps(g, acc2):
        i = 3 * g
        for u in range(3):
            b = u % NBUF
            bn = (u + 1) % NBUF
            bp = (u + 2) % NBUF
            start_ebuf(i + u + 2, bp)
            wait_ebuf(i + u + 1, bn)
            start_rows(bn)
            wait_rows(b)
            acc2 = compute(b, acc2)
        return acc2

    acc = lax.fori_loop(0, (STEPS - 2) // NBUF, three_steps, acc)
    wait_ebuf(STEPS - 1, 1)
    start_rows(1)
    wait_rows(0)
    acc = compute(0, acc)
    wait_rows(1)
    acc = compute(1, acc)

    accbuf[...] = acc
    pltpu.sync_copy(accbuf, out_hbm.at[wid])


def kernel(h, edge_index, edge_weight):
    src = edge_index[0].astype(jnp.int32).reshape(NW, STEPS, CHUNK)
    dst = edge_index[1].astype(jnp.int32).reshape(NW, STEPS, CHUNK)
    wfix = jnp.round(edge_weight * W_SCALE).astype(jnp.int32)
    packed = jnp.stack([src, dst, wfix.reshape(NW, STEPS, CHUNK)], axis=2)
    partials = _energy_kernel(packed, h)
    return jnp.sum(partials) / N_EDGES
